# Initial kernel scaffold; baseline (speedup 1.0000x reference)
import jax
import jax.numpy as jnp
from jax.experimental import pallas as pl


def kernel(logits):
    raise NotImplementedError("write your pallas kernel here")


# speedup vs baseline: 229.1591x; 229.1591x over previous
import jax
import jax.numpy as jnp
from jax.experimental import pallas as pl
from jax.experimental.pallas import tpu as pltpu

_TAU = 0.6
_EPS = 1e-06


def _body(logits_ref, delta_ref, tc_ref, yhat_ref, conf_s, y8_s, hist_s, tcb_s):
    p = pl.program_id(0)
    i = pl.program_id(1)
    C, ROWS, W = logits_ref.shape[1], logits_ref.shape[2], logits_ref.shape[3]

    @pl.when(p == 0)
    def _phase_a():
        @pl.when(i == 0)
        def _init():
            hist_s[...] = jnp.zeros(hist_s.shape, jnp.float32)

        m = logits_ref[0, 0]
        yv = jnp.zeros((ROWS, W), jnp.int32)
        for c in range(1, C):
            xc = logits_ref[0, c]
            gt = xc > m
            m = jnp.where(gt, xc, m)
            yv = jnp.where(gt, c, yv)
        s = jnp.exp(logits_ref[0, 0] - m)
        for c in range(1, C):
            s = s + jnp.exp(logits_ref[0, c] - m)
        conf = 1.0 / s
        high = conf > _TAU
        yhat_ref[0] = yv
        conf_s[i] = conf
        y8_s[i] = yv.astype(jnp.int8)
        for c in range(C):
            mc = jnp.logical_and(high, yv == c)
            part = jnp.sum(mc.astype(jnp.float32), axis=0, keepdims=True)
            hist_s[c : c + 1, :] = hist_s[c : c + 1, :] + part

    @pl.when(jnp.logical_and(p == 1, i == 0))
    def _compute_tc():
        sigma = jnp.sum(hist_s[...], axis=1, keepdims=True)
        mx = jnp.max(sigma)
        sh = sigma / jnp.maximum(mx, _EPS)
        tc = sh / (2.0 - jnp.minimum(sh, 1.0)) * _TAU
        tc_ref[...] = tc
        tcb_s[...] = jnp.broadcast_to(tc, tcb_s.shape)

    @pl.when(p == 1)
    def _phase_b():
        conf = conf_s[i]
        yv = y8_s[i].astype(jnp.int32)
        tmap = jnp.broadcast_to(tcb_s[0:1, :], (ROWS, W))
        for c in range(1, C):
            tmap = jnp.where(yv == c, jnp.broadcast_to(tcb_s[c : c + 1, :], (ROWS, W)), tmap)
        delta_ref[0] = conf > tmap


def kernel(logits):
    B, C, H, W = logits.shape
    ROWS = 128
    NB = H // ROWS
    N = B * NB

    def in_map(p, i):
        b = jnp.where(p == 0, i // NB, B - 1)
        h = jnp.where(p == 0, i % NB, NB - 1)
        return (b, 0, h, 0)

    def yhat_map(p, i):
        b = jnp.where(p == 0, i // NB, B - 1)
        h = jnp.where(p == 0, i % NB, NB - 1)
        return (b, h, 0)

    def delta_map(p, i):
        b = jnp.where(p == 0, 0, i // NB)
        h = jnp.where(p == 0, 0, i % NB)
        return (b, h, 0)

    delta, tc2d, yhat = pl.pallas_call(
        _body,
        grid=(2, N),
        in_specs=[pl.BlockSpec((1, C, ROWS, W), in_map)],
        out_specs=[
            pl.BlockSpec((1, ROWS, W), delta_map),
            pl.BlockSpec((C, 1), lambda p, i: (0, 0)),
            pl.BlockSpec((1, ROWS, W), yhat_map),
        ],
        out_shape=[
            jax.ShapeDtypeStruct((B, H, W), jnp.bool_),
            jax.ShapeDtypeStruct((C, 1), jnp.float32),
            jax.ShapeDtypeStruct((B, H, W), jnp.int32),
        ],
        scratch_shapes=[
            pltpu.VMEM((N, ROWS, W), jnp.float32),
            pltpu.VMEM((N, ROWS, W), jnp.int8),
            pltpu.VMEM((C, W), jnp.float32),
            pltpu.VMEM((C, W), jnp.float32),
        ],
    )(logits)
    return (delta, tc2d.reshape(C), yhat)
